# Initial kernel scaffold; baseline (speedup 1.0000x reference)
import jax
import jax.numpy as jnp
from jax.experimental import pallas as pl


def kernel(x, edge_index, W1, b1, g1, be1, W2, b2, g2, be2, W3, b3, g3, be3, Wg, a_src, a_dst, bg, ga, ba, linW, linb):
    raise NotImplementedError("write your pallas kernel here")


# speedup vs baseline: 59.6310x; 59.6310x over previous
import functools

import jax
import jax.numpy as jnp
from jax import lax
from jax.experimental import pallas as pl
from jax.experimental.pallas import tpu as pltpu
from jax.experimental.pallas import tpu_sc as plsc

N = 10000
E = 320000
D = 128
H = 4
FH = 32

NC = 2
NS = 16
NW = NC * NS
EPT = E // NW
CHUNK = 80
NCHUNK = EPT // CHUNK
NPAD = 10240
RPT = NPAD // NS


def _mesh():
    return plsc.VectorSubcoreMesh(core_axis_name="c", subcore_axis_name="s")


@functools.partial(
    pl.kernel,
    out_type=jax.ShapeDtypeStruct((NC, NPAD, 16), jnp.float32),
    mesh=_mesh(),
    compiler_params=pltpu.CompilerParams(use_tc_tiling_on_sc=False),
    scratch_types=[
        pltpu.VMEM((NCHUNK, CHUNK), jnp.int32),
        pltpu.VMEM((CHUNK, 16), jnp.float32),
        pltpu.VMEM_SHARED((NPAD, 16), jnp.float32),
    ],
)
def _deg_kernel(dst_hbm, ones_hbm, z_hbm, out_hbm, dst_v, ones_v, acc):
    c = lax.axis_index("c")
    s = lax.axis_index("s")
    wid = c * NS + s
    row0 = s * RPT
    pltpu.sync_copy(z_hbm.at[pl.ds(row0, RPT)], acc.at[pl.ds(row0, RPT)])
    pltpu.sync_copy(dst_hbm.at[wid], dst_v)
    pltpu.sync_copy(ones_hbm, ones_v)
    plsc.subcore_barrier()

    def body(j, carry):
        pltpu.sync_copy(ones_v, acc.at[dst_v.at[j]], add=True)
        return carry

    lax.fori_loop(0, NCHUNK, body, 0)
    plsc.subcore_barrier()
    pltpu.sync_copy(acc.at[pl.ds(row0, RPT)], out_hbm.at[c, pl.ds(row0, RPT)])


def _make_seg_sum(F):
    @functools.partial(
        pl.kernel,
        out_type=jax.ShapeDtypeStruct((NC, NPAD, F), jnp.float32),
        mesh=_mesh(),
        compiler_params=pltpu.CompilerParams(use_tc_tiling_on_sc=False),
        scratch_types=[
            pltpu.VMEM((NCHUNK, CHUNK), jnp.int32),
            pltpu.VMEM((NCHUNK, CHUNK), jnp.int32),
            pltpu.VMEM((CHUNK, F), jnp.float32),
            pltpu.VMEM((CHUNK, F), jnp.float32),
            pltpu.VMEM_SHARED((NPAD, F), jnp.float32),
            pltpu.SemaphoreType.DMA,
            pltpu.SemaphoreType.DMA,
        ],
    )
    def seg_sum(hs_hbm, src_hbm, dst_hbm, z_hbm, out_hbm,
                src_v, dst_v, buf0, buf1, acc, sem0, sem1):
        c = lax.axis_index("c")
        s = lax.axis_index("s")
        wid = c * NS + s
        row0 = s * RPT
        pltpu.sync_copy(z_hbm.at[pl.ds(row0, RPT)], acc.at[pl.ds(row0, RPT)])
        pltpu.sync_copy(src_hbm.at[wid], src_v)
        pltpu.sync_copy(dst_hbm.at[wid], dst_v)
        plsc.subcore_barrier()

        bufs = (buf0, buf1)
        sems = (sem0, sem1)
        pltpu.async_copy(hs_hbm.at[src_v.at[0]], buf0, sem0)

        def body(j, carry):
            for par in range(2):
                @pl.when(j % 2 == par)
                def _():
                    @pl.when(j + 1 < NCHUNK)
                    def _():
                        pltpu.async_copy(
                            hs_hbm.at[src_v.at[j + 1]], bufs[1 - par],
                            sems[1 - par])
                    pltpu.make_async_copy(
                        hs_hbm.at[src_v.at[j]], bufs[par], sems[par]).wait()
                    pltpu.sync_copy(bufs[par], acc.at[dst_v.at[j]], add=True)
            return carry

        lax.fori_loop(0, NCHUNK, body, 0)
        plsc.subcore_barrier()
        pltpu.sync_copy(acc.at[pl.ds(row0, RPT)],
                        out_hbm.at[c, pl.ds(row0, RPT)])

    return seg_sum


_seg_sum_16 = _make_seg_sum(16)
_seg_sum_64 = _make_seg_sum(64)


@functools.partial(
    pl.kernel,
    out_type=(
        jax.ShapeDtypeStruct((NC, NPAD, D), jnp.float32),
        jax.ShapeDtypeStruct((NC, NPAD, 16), jnp.float32),
    ),
    mesh=_mesh(),
    compiler_params=pltpu.CompilerParams(use_tc_tiling_on_sc=False),
    scratch_types=[
        pltpu.VMEM((NCHUNK, CHUNK), jnp.int32),
        pltpu.VMEM((NCHUNK, CHUNK), jnp.int32),
        pltpu.VMEM((CHUNK, 16), jnp.float32),
        pltpu.VMEM((CHUNK, 16), jnp.float32),
        pltpu.VMEM((CHUNK, D), jnp.float32),
        pltpu.VMEM((16,), jnp.float32),
        pltpu.VMEM_SHARED((NPAD, D), jnp.float32),
        pltpu.VMEM_SHARED((NPAD, 16), jnp.float32),
        pltpu.SemaphoreType.DMA,
        pltpu.SemaphoreType.DMA,
        pltpu.SemaphoreType.DMA,
    ],
)
def _gat_kernel(hg_hbm, sa_hbm, da_hbm, src_hbm, dst_hbm,
                mt_hbm, z128_hbm, z16_hbm,
                num_hbm, den_hbm,
                src_v, dst_v, sabuf, dabuf, rowbuf, mtv,
                acc_num, acc_den, sem_h, sem_s, sem_d):
    c = lax.axis_index("c")
    s = lax.axis_index("s")
    wid = c * NS + s
    row0 = s * RPT
    pltpu.sync_copy(z128_hbm.at[pl.ds(row0, RPT)],
                    acc_num.at[pl.ds(row0, RPT)])
    pltpu.sync_copy(z16_hbm.at[pl.ds(row0, RPT)],
                    acc_den.at[pl.ds(row0, RPT)])
    pltpu.sync_copy(mt_hbm, mtv)
    pltpu.sync_copy(src_hbm.at[wid], src_v)
    pltpu.sync_copy(dst_hbm.at[wid], dst_v)
    plsc.subcore_barrier()
    mvec = mtv[...]

    def body(j, carry):
        cp_h = pltpu.async_copy(hg_hbm.at[src_v.at[j]], rowbuf, sem_h)
        cp_s = pltpu.async_copy(sa_hbm.at[src_v.at[j]], sabuf, sem_s)
        cp_d = pltpu.async_copy(da_hbm.at[dst_v.at[j]], dabuf, sem_d)
        cp_s.wait()
        cp_d.wait()

        def ee_body(r, carry2):
            v = sabuf[r] + dabuf[r]
            v = jnp.where(v > 0.0, v, v * jnp.float32(0.2))
            sabuf[r] = jnp.exp(v - mvec)
            return carry2

        lax.fori_loop(0, CHUNK, ee_body, 0)
        pltpu.sync_copy(sabuf, acc_den.at[dst_v.at[j]], add=True)
        cp_h.wait()

        def sc_body(r, carry2):
            eerow = sabuf[r]
            for j8 in range(D // 16):
                rowbuf[r, pl.ds(j8 * 16, 16)] = (
                    rowbuf[r, pl.ds(j8 * 16, 16)] * eerow[j8 // 2])
            return carry2

        lax.fori_loop(0, CHUNK, sc_body, 0)
        pltpu.sync_copy(rowbuf, acc_num.at[dst_v.at[j]], add=True)
        return carry

    lax.fori_loop(0, NCHUNK, body, 0)
    plsc.subcore_barrier()
    pltpu.sync_copy(acc_num.at[pl.ds(row0, RPT)],
                    num_hbm.at[c, pl.ds(row0, RPT)])
    pltpu.sync_copy(acc_den.at[pl.ds(row0, RPT)],
                    den_hbm.at[c, pl.ds(row0, RPT)])


def _bn(x, g, b, eps=1e-5):
    m = x.mean(axis=0)
    v = x.var(axis=0)
    return g * (x - m) / jnp.sqrt(v + eps) + b


def kernel(x, edge_index, W1, b1, g1, be1, W2, b2, g2, be2, W3, b3, g3, be3,
           Wg, a_src, a_dst, bg, ga, ba, linW, linb):
    src3 = edge_index[0].reshape(NW, NCHUNK, CHUNK)
    dst3 = edge_index[1].reshape(NW, NCHUNK, CHUNK)
    ones16 = jnp.ones((CHUNK, 16), jnp.float32)
    z16 = jnp.zeros((NPAD, 16), jnp.float32)
    z64 = jnp.zeros((NPAD, 64), jnp.float32)
    z128 = jnp.zeros((NPAD, D), jnp.float32)

    degp = _deg_kernel(dst3, ones16, z16)
    deg = degp[0, :N, 0] + degp[1, :N, 0] + 1.0
    dis = lax.rsqrt(deg)[:, None]

    def gcn(h, W, b, g, be, seg):
        hs = dis * (h @ W)
        accp = seg(hs, src3, dst3, z16 if W.shape[1] == 16 else z64)
        acc = accp[0, :N] + accp[1, :N] + hs
        return jax.nn.relu(_bn(dis * acc + b, g, be))

    h1 = gcn(x, W1, b1, g1, be1, _seg_sum_16)
    h2 = gcn(h1, W2, b2, g2, be2, _seg_sum_64)
    h3 = gcn(h2, W3, b3, g3, be3, _seg_sum_64)
    h = h2 + h3

    hg = h @ Wg
    hg3 = hg.reshape(N, H, FH)
    asrc = (hg3 * a_src[None]).sum(-1)
    adst = (hg3 * a_dst[None]).sum(-1)
    m = asrc.max(axis=0) + adst.max(axis=0)
    mm = jnp.where(m > 0.0, m, m * 0.2)
    sa16 = jnp.pad(asrc, ((0, 0), (0, 12)))
    da16 = jnp.pad(adst, ((0, 0), (0, 12)))
    mt16 = jnp.pad(mm, (0, 12))

    nump, denp = _gat_kernel(hg, sa16, da16, src3, dst3, mt16, z128, z16)
    ee_self = jnp.exp(jnp.where(asrc + adst > 0.0, asrc + adst,
                                (asrc + adst) * 0.2) - mm[None])
    den = denp[0, :N, :H] + denp[1, :N, :H] + ee_self
    num = (nump[0, :N] + nump[1, :N]
           + (hg3 * ee_self[:, :, None]).reshape(N, D))
    gat = num / jnp.repeat(den + 1e-16, FH, axis=1) + bg
    h4 = jax.nn.elu(_bn(gat, ga, ba))
    return h4 @ linW + linb
